# baseline (speedup 1.0000x reference)
import functools

import jax
import jax.numpy as jnp
from jax import lax
from jax.experimental import pallas as pl
from jax.experimental.pallas import tpu as pltpu
from jax.experimental.pallas import tpu_sc as plsc

H, G, E, F = 50000, 100000, 800000, 64
NC, NS = 2, 16
NW = NC * NS
CHUNKP = 25088
RPT = CHUNKP // NS
DUMMY = CHUNKP
ACC_ROWS = CHUNKP + 8
HP = 2 * CHUNKP
GP = 4 * CHUNKP

KB = 64
NBB = 25
FB = NBB * KB
SLOT = -(-25008 // FB) * FB
SROW = SLOT // KB
RING = 6

EPR = 1563
E2R = NW * EPR
CBLK = [(i * 200, 200) for i in range(7)] + [(1400, 163)]


def _compaction(rows_hbm, cols_hbm, dummy_hbm,
                src_h, dst_h, cnt_h, src_g, dst_g, cnt_g,
                sbuf, dbuf, stg_s0, stg_d0, stg_s1, stg_d1, cntbuf):
    w = lax.axis_index("c") * NS + lax.axis_index("s")
    row0 = w * EPR

    passes = [
        (rows_hbm, cols_hbm, (0, 1), src_h, dst_h, cnt_h),
        (cols_hbm, rows_hbm, (0, 1), src_g, dst_g, cnt_g),
        (cols_hbm, rows_hbm, (2, 3), src_g, dst_g, cnt_g),
    ]
    for darr, sarr, chunks, src_o, dst_o, cnt_o in passes:
        for stg in (stg_s0, stg_d0, stg_s1, stg_d1):
            for fb in range(SROW // NBB):
                pltpu.sync_copy(dummy_hbm, stg.at[pl.ds(fb * NBB, NBB)])

        zero = jnp.zeros((16,), jnp.int32)
        offs = (zero, zero)
        for boff, nr in CBLK:
            pltpu.sync_copy(darr.at[pl.ds(row0 + boff, nr)],
                            dbuf.at[pl.ds(0, nr)])
            pltpu.sync_copy(sarr.at[pl.ds(row0 + boff, nr)],
                            sbuf.at[pl.ds(0, nr)])

            @pl.loop(0, nr, init_carry=offs)
            def _scan(j, carry):
                off0, off1 = carry
                d = dbuf[j]
                s = sbuf[j]
                outs = []
                for ci, (off, stgs, stgd) in enumerate(
                        [(off0, stg_s0, stg_d0), (off1, stg_s1, stg_d1)]):
                    loc = d - chunks[ci] * CHUNKP
                    m = (loc >= 0) & (loc < CHUNKP)
                    mi = jnp.where(m, 1, 0)
                    pos = off + plsc.cumsum(mi) - mi
                    prow = lax.shift_right_logical(pos, KB.bit_length() - 1)
                    plane = pos & (KB - 1)
                    plsc.store_scatter(stgd, [prow, plane], loc, mask=m)
                    plsc.store_scatter(stgs, [prow, plane], s, mask=m)
                    outs.append(off + plsc.all_reduce_population_count(m))
                return tuple(outs)

            offs = _scan

        for ci, (off, stgs, stgd) in enumerate(
                [(offs[0], stg_s0, stg_d0), (offs[1], stg_s1, stg_d1)]):
            ck = chunks[ci]
            cntbuf[...] = off
            pltpu.sync_copy(cntbuf, cnt_o.at[ck, w])
            cnt = cntbuf[pl.ds(0, 16)][0]
            nfb = (cnt + FB - 1) // FB

            @pl.loop(0, nfb)
            def _flush(fb):
                r = fb * NBB
                pltpu.sync_copy(stgd.at[pl.ds(r, NBB)],
                                dst_o.at[ck, w, pl.ds(r, NBB)])
                pltpu.sync_copy(stgs.at[pl.ds(r, NBB)],
                                src_o.at[ck, w, pl.ds(r, NBB)])


def _spmm_body(cpc, src_l, dst_l, cnt_l, x_hbm, init_hbm, out_hbm,
               srcblk, idxblk, rowsbuf, cbuf, gsem, ssem, acc):
    core = lax.axis_index("c")
    sub = lax.axis_index("s")
    for k in range(cpc):
        chunk = core * cpc + k
        gb = chunk * CHUNKP + sub * RPT
        lb = sub * RPT
        pltpu.sync_copy(init_hbm.at[pl.ds(gb, RPT)], acc.at[pl.ds(lb, RPT)])
        plsc.subcore_barrier()

        for li in range(2):
            w = sub * 2 + li
            pltpu.sync_copy(cnt_l.at[chunk, w], cbuf)
            cnt = cbuf[pl.ds(0, 16)][0]
            nfb = (cnt + FB - 1) // FB

            @pl.loop(0, nfb)
            def _flush_block(fb):
                r = fb * NBB
                pltpu.sync_copy(src_l.at[chunk, w, pl.ds(r, NBB)], srcblk)
                pltpu.sync_copy(dst_l.at[chunk, w, pl.ds(r, NBB)], idxblk)

                def gat(j, b):
                    return pltpu.make_async_copy(x_hbm.at[srcblk.at[j]],
                                                 rowsbuf.at[b], gsem)

                def sca(j, b):
                    return pltpu.make_async_copy(rowsbuf.at[b],
                                                 acc.at[idxblk.at[j]], ssem)

                gat(0, 0).start()
                gat(1, 1).start()
                gat(2, 2).start()

                @pl.loop(0, NBB)
                def _stream(j):
                    b = lax.rem(j, RING)
                    gat(j, b).wait()
                    pltpu.async_copy(rowsbuf.at[b], acc.at[idxblk.at[j]],
                                     ssem, add=True)

                    @pl.when(j >= 3)
                    def _():
                        sca(j, b).wait()

                    @pl.when(j + 3 < NBB)
                    def _():
                        pltpu.async_copy(x_hbm.at[srcblk.at[j + 3]],
                                         rowsbuf.at[lax.rem(j + 3, RING)],
                                         gsem)

                sca(0, 0).wait()
                sca(0, 0).wait()
                sca(0, 0).wait()

        plsc.subcore_barrier()
        pltpu.sync_copy(acc.at[pl.ds(lb, RPT)], out_hbm.at[pl.ds(gb, RPT)])
        if k + 1 < cpc:
            plsc.subcore_barrier()


_MESH = plsc.VectorSubcoreMesh(core_axis_name="c", subcore_axis_name="s",
                               num_cores=NC, num_subcores=NS)
_PARAMS = pltpu.CompilerParams(use_tc_tiling_on_sc=False,
                               needs_layout_passes=False)

_i32 = jnp.int32
_compact = pl.kernel(
    _compaction,
    out_type=(
        jax.ShapeDtypeStruct((2, NW, SROW, KB), _i32),
        jax.ShapeDtypeStruct((2, NW, SROW, KB), _i32),
        jax.ShapeDtypeStruct((2, NW, 16), _i32),
        jax.ShapeDtypeStruct((4, NW, SROW, KB), _i32),
        jax.ShapeDtypeStruct((4, NW, SROW, KB), _i32),
        jax.ShapeDtypeStruct((4, NW, 16), _i32),
    ),
    mesh=_MESH,
    scratch_types=[
        pltpu.VMEM((200, 16), _i32),
        pltpu.VMEM((200, 16), _i32),
        pltpu.VMEM((SROW, KB), _i32),
        pltpu.VMEM((SROW, KB), _i32),
        pltpu.VMEM((SROW, KB), _i32),
        pltpu.VMEM((SROW, KB), _i32),
        pltpu.VMEM((16,), _i32),
    ],
    compiler_params=_PARAMS,
)


def _make_spmm(ndst_p):
    cpc = ndst_p // CHUNKP // NC
    f32 = jnp.float32
    return pl.kernel(
        functools.partial(_spmm_body, cpc),
        out_type=jax.ShapeDtypeStruct((ndst_p, F), f32),
        mesh=_MESH,
        scratch_types=[
            pltpu.VMEM((NBB, KB), _i32),
            pltpu.VMEM((NBB, KB), _i32),
            pltpu.VMEM((RING, KB, F), f32),
            pltpu.VMEM((16,), _i32),
            pltpu.SemaphoreType.DMA,
            pltpu.SemaphoreType.DMA,
            pltpu.VMEM_SHARED((ACC_ROWS, F), f32),
        ],
        compiler_params=_PARAMS,
    )


_spmm_h = _make_spmm(HP)
_spmm_g = _make_spmm(GP)


def kernel(herb, gene_i, gene_j, embed_herb, embed_gene,
           edge_rows, edge_cols, edge_vals, d_i, d_j):
    npad = E2R * 16 - E
    rows = jnp.pad(edge_rows.astype(_i32), (0, npad),
                   constant_values=-1).reshape(E2R, 16)
    cols = jnp.pad(edge_cols.astype(_i32), (0, npad),
                   constant_values=-1).reshape(E2R, 16)
    dummy = jnp.full((NBB, KB), DUMMY, _i32)

    src_h, dst_h, cnt_h, src_g, dst_g, cnt_g = _compact(rows, cols, dummy)

    sh = jnp.sqrt(d_i)
    sg = jnp.sqrt(d_j)
    inv_sh = 1.0 / sh
    inv_sg = 1.0 / sg

    def padp(x, n):
        return jnp.pad(x, ((0, n - x.shape[0]), (0, 0)))

    h_hat = padp(embed_herb * sh, HP)
    g_hat = padp(embed_gene * sg, GP)
    dip = padp(d_i, HP)
    djp = padp(d_j, GP)

    hs, gs = [], []
    for _ in range(3):
        u_h = _spmm_h(src_h, dst_h, cnt_h, g_hat, h_hat)
        u_g = _spmm_g(src_g, dst_g, cnt_g, h_hat, g_hat)
        h_hat = dip * u_h
        g_hat = djp * u_g
        hs.append(h_hat[:H] * inv_sh)
        gs.append(g_hat[:G] * inv_sg)

    gcn_herbs = jnp.concatenate([embed_herb] + hs, axis=-1)
    gcn_genes = jnp.concatenate([embed_gene] + gs, axis=-1)
    return (gcn_herbs, gcn_genes)


# speedup vs baseline: 1.0156x; 1.0156x over previous
import functools

import jax
import jax.numpy as jnp
from jax import lax
from jax.experimental import pallas as pl
from jax.experimental.pallas import tpu as pltpu
from jax.experimental.pallas import tpu_sc as plsc

H, G, E, F = 50000, 100000, 800000, 64
NC, NS = 2, 16
NW = NC * NS
CHUNKP = 25088
RPT = CHUNKP // NS
DUMMY = CHUNKP
ACC_ROWS = CHUNKP + 8
HP = 2 * CHUNKP
GP = 4 * CHUNKP

KB = 64
NBB = 25
FB = NBB * KB
SLOT = -(-25008 // FB) * FB
SROW = SLOT // KB
RING = 6

EPR = 1563
E2R = NW * EPR
CBLK = [(i * 200, 200) for i in range(7)] + [(1400, 163)]


def _compaction(rows_hbm, cols_hbm, dummy_hbm,
                src_h, dst_h, cnt_h, src_g, dst_g, cnt_g,
                sbuf, dbuf, stg_s0, stg_d0, stg_s1, stg_d1, cntbuf):
    w = lax.axis_index("c") * NS + lax.axis_index("s")
    row0 = w * EPR

    passes = [
        (rows_hbm, cols_hbm, (0, 1), src_h, dst_h, cnt_h),
        (cols_hbm, rows_hbm, (0, 1), src_g, dst_g, cnt_g),
        (cols_hbm, rows_hbm, (2, 3), src_g, dst_g, cnt_g),
    ]
    for darr, sarr, chunks, src_o, dst_o, cnt_o in passes:
        for stg in (stg_s0, stg_d0, stg_s1, stg_d1):
            for fb in range(SROW // NBB):
                pltpu.sync_copy(dummy_hbm, stg.at[pl.ds(fb * NBB, NBB)])

        zero = jnp.zeros((16,), jnp.int32)
        offs = (zero, zero)
        for boff, nr in CBLK:
            pltpu.sync_copy(darr.at[pl.ds(row0 + boff, nr)],
                            dbuf.at[pl.ds(0, nr)])
            pltpu.sync_copy(sarr.at[pl.ds(row0 + boff, nr)],
                            sbuf.at[pl.ds(0, nr)])

            @pl.loop(0, nr, init_carry=offs)
            def _scan(j, carry):
                off0, off1 = carry
                d = dbuf[j]
                s = sbuf[j]
                outs = []
                for ci, (off, stgs, stgd) in enumerate(
                        [(off0, stg_s0, stg_d0), (off1, stg_s1, stg_d1)]):
                    loc = d - chunks[ci] * CHUNKP
                    m = (loc >= 0) & (loc < CHUNKP)
                    mi = jnp.where(m, 1, 0)
                    pos = off + plsc.cumsum(mi) - mi
                    prow = lax.shift_right_logical(pos, KB.bit_length() - 1)
                    plane = pos & (KB - 1)
                    plsc.store_scatter(stgd, [prow, plane], loc, mask=m)
                    plsc.store_scatter(stgs, [prow, plane], s, mask=m)
                    outs.append(off + plsc.all_reduce_population_count(m))
                return tuple(outs)

            offs = _scan

        for ci, (off, stgs, stgd) in enumerate(
                [(offs[0], stg_s0, stg_d0), (offs[1], stg_s1, stg_d1)]):
            ck = chunks[ci]
            cntbuf[...] = off
            pltpu.sync_copy(cntbuf, cnt_o.at[ck, w])
            cnt = cntbuf[pl.ds(0, 16)][0]
            nfb = (cnt + FB - 1) // FB

            @pl.loop(0, nfb)
            def _flush(fb):
                r = fb * NBB
                pltpu.sync_copy(stgd.at[pl.ds(r, NBB)],
                                dst_o.at[ck, w, pl.ds(r, NBB)])
                pltpu.sync_copy(stgs.at[pl.ds(r, NBB)],
                                src_o.at[ck, w, pl.ds(r, NBB)])


def _spmm_body(cpc, src_l, dst_l, cnt_l, x_hbm, init_hbm, out_hbm,
               srcblk, idxblk, rowsbuf, cbuf, gsem, ssem, acc):
    core = lax.axis_index("c")
    sub = lax.axis_index("s")
    for k in range(cpc):
        chunk = core * cpc + k
        gb = chunk * CHUNKP + sub * RPT
        lb = sub * RPT
        pltpu.sync_copy(init_hbm.at[pl.ds(gb, RPT)], acc.at[pl.ds(lb, RPT)])
        plsc.subcore_barrier()

        for li in range(2):
            w = sub * 2 + li
            pltpu.sync_copy(cnt_l.at[chunk, w], cbuf)
            cnt = cbuf[pl.ds(0, 16)][0]
            nfb = (cnt + FB - 1) // FB

            @pl.loop(0, nfb)
            def _flush_block(fb):
                r = fb * NBB
                pltpu.sync_copy(src_l.at[chunk, w, pl.ds(r, NBB)], srcblk)
                pltpu.sync_copy(dst_l.at[chunk, w, pl.ds(r, NBB)], idxblk)

                def gat(j, b):
                    return pltpu.make_async_copy(x_hbm.at[srcblk.at[j]],
                                                 rowsbuf.at[b], gsem)

                def sca(j, b):
                    return pltpu.make_async_copy(rowsbuf.at[b],
                                                 acc.at[idxblk.at[j]], ssem)

                gat(0, 0).start()
                gat(1, 1).start()
                gat(2, 2).start()
                gat(3, 3).start()

                @pl.loop(0, NBB)
                def _stream(j):
                    b = lax.rem(j, RING)
                    gat(j, b).wait()
                    pltpu.async_copy(rowsbuf.at[b], acc.at[idxblk.at[j]],
                                     ssem, add=True)

                    @pl.when(j >= 2)
                    def _():
                        sca(j, b).wait()

                    @pl.when(j + 4 < NBB)
                    def _():
                        pltpu.async_copy(x_hbm.at[srcblk.at[j + 4]],
                                         rowsbuf.at[lax.rem(j + 4, RING)],
                                         gsem)

                sca(0, 0).wait()
                sca(0, 0).wait()

        plsc.subcore_barrier()
        pltpu.sync_copy(acc.at[pl.ds(lb, RPT)], out_hbm.at[pl.ds(gb, RPT)])
        if k + 1 < cpc:
            plsc.subcore_barrier()


_MESH = plsc.VectorSubcoreMesh(core_axis_name="c", subcore_axis_name="s",
                               num_cores=NC, num_subcores=NS)
_PARAMS = pltpu.CompilerParams(use_tc_tiling_on_sc=False,
                               needs_layout_passes=False)

_i32 = jnp.int32
_compact = pl.kernel(
    _compaction,
    out_type=(
        jax.ShapeDtypeStruct((2, NW, SROW, KB), _i32),
        jax.ShapeDtypeStruct((2, NW, SROW, KB), _i32),
        jax.ShapeDtypeStruct((2, NW, 16), _i32),
        jax.ShapeDtypeStruct((4, NW, SROW, KB), _i32),
        jax.ShapeDtypeStruct((4, NW, SROW, KB), _i32),
        jax.ShapeDtypeStruct((4, NW, 16), _i32),
    ),
    mesh=_MESH,
    scratch_types=[
        pltpu.VMEM((200, 16), _i32),
        pltpu.VMEM((200, 16), _i32),
        pltpu.VMEM((SROW, KB), _i32),
        pltpu.VMEM((SROW, KB), _i32),
        pltpu.VMEM((SROW, KB), _i32),
        pltpu.VMEM((SROW, KB), _i32),
        pltpu.VMEM((16,), _i32),
    ],
    compiler_params=_PARAMS,
)


def _make_spmm(ndst_p):
    cpc = ndst_p // CHUNKP // NC
    f32 = jnp.float32
    return pl.kernel(
        functools.partial(_spmm_body, cpc),
        out_type=jax.ShapeDtypeStruct((ndst_p, F), f32),
        mesh=_MESH,
        scratch_types=[
            pltpu.VMEM((NBB, KB), _i32),
            pltpu.VMEM((NBB, KB), _i32),
            pltpu.VMEM((RING, KB, F), f32),
            pltpu.VMEM((16,), _i32),
            pltpu.SemaphoreType.DMA,
            pltpu.SemaphoreType.DMA,
            pltpu.VMEM_SHARED((ACC_ROWS, F), f32),
        ],
        compiler_params=_PARAMS,
    )


_spmm_h = _make_spmm(HP)
_spmm_g = _make_spmm(GP)


def kernel(herb, gene_i, gene_j, embed_herb, embed_gene,
           edge_rows, edge_cols, edge_vals, d_i, d_j):
    npad = E2R * 16 - E
    rows = jnp.pad(edge_rows.astype(_i32), (0, npad),
                   constant_values=-1).reshape(E2R, 16)
    cols = jnp.pad(edge_cols.astype(_i32), (0, npad),
                   constant_values=-1).reshape(E2R, 16)
    dummy = jnp.full((NBB, KB), DUMMY, _i32)

    src_h, dst_h, cnt_h, src_g, dst_g, cnt_g = _compact(rows, cols, dummy)

    sh = jnp.sqrt(d_i)
    sg = jnp.sqrt(d_j)
    inv_sh = 1.0 / sh
    inv_sg = 1.0 / sg

    def padp(x, n):
        return jnp.pad(x, ((0, n - x.shape[0]), (0, 0)))

    h_hat = padp(embed_herb * sh, HP)
    g_hat = padp(embed_gene * sg, GP)
    dip = padp(d_i, HP)
    djp = padp(d_j, GP)

    hs, gs = [], []
    for _ in range(3):
        u_h = _spmm_h(src_h, dst_h, cnt_h, g_hat, h_hat)
        u_g = _spmm_g(src_g, dst_g, cnt_g, h_hat, g_hat)
        h_hat = dip * u_h
        g_hat = djp * u_g
        hs.append(h_hat[:H] * inv_sh)
        gs.append(g_hat[:G] * inv_sg)

    gcn_herbs = jnp.concatenate([embed_herb] + hs, axis=-1)
    gcn_genes = jnp.concatenate([embed_gene] + gs, axis=-1)
    return (gcn_herbs, gcn_genes)
